# Initial kernel scaffold; baseline (speedup 1.0000x reference)
import jax
import jax.numpy as jnp
from jax.experimental import pallas as pl


def kernel(x, adjacency, motif_adjacency, W_gcn, b_gcn, W_lin, b_lin):
    raise NotImplementedError("write your pallas kernel here")


# speedup vs baseline: 1.0725x; 1.0725x over previous
import math

import jax
import jax.numpy as jnp
import numpy as np
from jax import lax
from jax.experimental import pallas as pl
from jax.experimental.pallas import tpu as pltpu
from jax.experimental.pallas import tpu_sc as plsc

_N = 8192
_C = 128
_K = 4096
_NC = 2
_NS = 16
_NW = _NC * _NS
_RPW = _K // _NW
_RB = 8
_MININT = np.int32(-2147483648)


def _sc_body(keys_hbm, h_hbm, adj_hbm, motif_hbm,
             topidx_hbm, xpool_hbm, adjp_hbm, motifp_hbm,
             key_v, idx_v, xrows_v, rows_v, orow_v, sem):
    cid = lax.axis_index("c")
    sid = lax.axis_index("s")
    wid = sid * _NC + cid
    base = wid * _RPW

    ones16 = jnp.ones((16,), jnp.int32)
    zeros16 = jnp.zeros((16,), jnp.int32)
    iota16 = lax.iota(jnp.int32, 16)

    pltpu.sync_copy(keys_hbm, key_v)

    def _count(ts, strict):
        def body(i, acc):
            for u in range(8):
                mvec = key_v[pl.ds((i * 8 + u) * 16, 16)]
                msk = (mvec > ts) if strict else (mvec >= ts)
                acc = acc + jnp.where(msk, ones16, zeros16)
            return acc
        accv = lax.fori_loop(0, _N // (16 * 8), body, zeros16)
        return jnp.sum(accv)

    def _bis(b, tu):
        cand = tu | (jnp.int32(1) << (jnp.int32(31) - b))
        cnt = _count(cand ^ _MININT, False)
        return jnp.where(cnt >= _K, cand, tu)

    tu = lax.fori_loop(0, 32, _bis, jnp.int32(0))
    ts = tu ^ _MININT
    need = jnp.int32(_K) - _count(ts, True)

    def _sel(i, carry):
        nsel, neq = carry
        mvec = key_v[pl.ds(i * 16, 16)]
        gt = mvec > ts
        eq = mvec == ts
        eq01 = jnp.where(eq, ones16, zeros16)
        eqrank = neq + plsc.cumsum(eq01) - eq01
        sel = gt | (eq & (eqrank < need))
        sel01 = jnp.where(sel, ones16, zeros16)
        pos = nsel + plsc.cumsum(sel01) - sel01
        pos = jnp.where(sel, pos, zeros16)
        plsc.store_scatter(idx_v, [pos], i * 16 + iota16, mask=sel)
        return (nsel + jnp.sum(sel01), neq + jnp.sum(eq01))

    lax.fori_loop(0, _N // 16, _sel, (jnp.int32(0), jnp.int32(0)))

    @pl.when(wid == 0)
    def _():
        pltpu.sync_copy(idx_v, topidx_hbm)

    pltpu.async_copy(h_hbm.at[idx_v.at[pl.ds(base, _RPW)]], xrows_v, sem).wait()
    pltpu.sync_copy(xrows_v, xpool_hbm.at[pl.ds(base, _RPW)])

    def _pool(tab_hbm, out_hbm):
        def batch(bi, _):
            r0 = base + bi * _RB
            pltpu.async_copy(tab_hbm.at[idx_v.at[pl.ds(r0, _RB)]],
                             rows_v, sem).wait()

            def cols(ci, _):
                cvec = idx_v[pl.ds(ci * 16, 16)]
                for r in range(_RB):
                    vals = plsc.load_gather(
                        rows_v, [jnp.full((16,), r, jnp.int32), cvec])
                    orow_v[pl.ds(r * _K + ci * 16, 16)] = vals
                return 0

            lax.fori_loop(0, _K // 16, cols, 0, unroll=4)
            pltpu.sync_copy(orow_v, out_hbm.at[pl.ds(r0 * _K, _RB * _K)])
            return 0

        lax.fori_loop(0, _RPW // _RB, batch, 0)

    _pool(adj_hbm, adjp_hbm)
    _pool(motif_hbm, motifp_hbm)


def _sc_select_pool(keys, h, adjacency, motif):
    mesh = plsc.VectorSubcoreMesh(core_axis_name="c", subcore_axis_name="s")
    f = pl.kernel(
        _sc_body,
        out_type=(
            jax.ShapeDtypeStruct((_K,), jnp.int32),
            jax.ShapeDtypeStruct((_K, _C), jnp.float32),
            jax.ShapeDtypeStruct((_K * _K,), jnp.float32),
            jax.ShapeDtypeStruct((_K * _K,), jnp.float32),
        ),
        mesh=mesh,
        compiler_params=pltpu.CompilerParams(needs_layout_passes=False),
        scratch_types=[
            pltpu.VMEM((_N,), jnp.int32),
            pltpu.VMEM((_K,), jnp.int32),
            pltpu.VMEM((_RPW, _C), jnp.float32),
            pltpu.VMEM((_RB, _N), jnp.float32),
            pltpu.VMEM((_RB * _K,), jnp.float32),
            pltpu.SemaphoreType.DMA,
        ],
    )
    return f(keys, h, adjacency, motif)


def kernel(x, adjacency, motif_adjacency, W_gcn, b_gcn, W_lin, b_lin):
    num_nodes = x.shape[0]
    A = motif_adjacency + jnp.eye(num_nodes, dtype=x.dtype)
    deg = A.sum(axis=0)
    dinv = jnp.where(deg > 0, 1.0 / jnp.sqrt(deg), 0.0)
    norm = dinv[:, None] * A * dinv[None, :]
    xw = x @ W_gcn
    agg = norm.T @ xw + b_gcn
    h = jnp.tanh(agg)
    scores = (h @ W_lin + b_lin).squeeze(-1)

    bits = lax.bitcast_convert_type(scores, jnp.int32)
    keys = bits ^ ((bits >> 31) & jnp.int32(0x7FFFFFFF))

    top_idx, x_pool, adjp_flat, motifp_flat = _sc_select_pool(
        keys, h, adjacency, motif_adjacency)
    return (x_pool, adjp_flat.reshape(_K, _K), motifp_flat.reshape(_K, _K),
            scores, top_idx)


# speedup vs baseline: 1.1678x; 1.0889x over previous
import math

import jax
import jax.numpy as jnp
import numpy as np
from jax import lax
from jax.experimental import pallas as pl
from jax.experimental.pallas import tpu as pltpu
from jax.experimental.pallas import tpu_sc as plsc

_N = 8192
_C = 128
_K = 4096
_NC = 2
_NS = 16
_NW = _NC * _NS
_RPW = _K // _NW
_RB = 4
_NB = _RPW // _RB
_MININT = np.int32(-2147483648)


def _sc_body(keys_hbm, h_hbm, adj_hbm, motif_hbm,
             topidx_hbm, xpool_hbm, adjp_hbm, motifp_hbm,
             key_v, idx_v, xrows_v, pad_v, rows0_v, rows1_v, orow0_v, orow1_v,
             isem0, isem1, osem0, osem1, xsem):
    cid = lax.axis_index("c")
    sid = lax.axis_index("s")
    wid = sid * _NC + cid
    base = wid * _RPW

    ones16 = jnp.ones((16,), jnp.int32)
    zeros16 = jnp.zeros((16,), jnp.int32)
    iota16 = lax.iota(jnp.int32, 16)

    pltpu.sync_copy(keys_hbm, key_v)

    def _count(ts, strict):
        def body(i, acc):
            for u in range(8):
                mvec = key_v[pl.ds((i * 8 + u) * 16, 16)]
                msk = (mvec > ts) if strict else (mvec >= ts)
                acc = acc + jnp.where(msk, ones16, zeros16)
            return acc
        accv = lax.fori_loop(0, _N // (16 * 8), body, zeros16)
        return jnp.sum(accv)

    def _bis(b, tu):
        cand = tu | (jnp.int32(1) << (jnp.int32(31) - b))
        cnt = _count(cand ^ _MININT, False)
        return jnp.where(cnt >= _K, cand, tu)

    tu = lax.fori_loop(0, 32, _bis, jnp.int32(0))
    ts = tu ^ _MININT
    need = jnp.int32(_K) - _count(ts, True)

    def _sel(i, carry):
        nsel, neq = carry
        mvec = key_v[pl.ds(i * 16, 16)]
        gt = mvec > ts
        eq = mvec == ts
        eq01 = jnp.where(eq, ones16, zeros16)
        eqrank = neq + plsc.cumsum(eq01) - eq01
        sel = gt | (eq & (eqrank < need))
        sel01 = jnp.where(sel, ones16, zeros16)
        pos = nsel + plsc.cumsum(sel01) - sel01
        pos = jnp.where(sel, pos, zeros16)
        plsc.store_scatter(idx_v, [pos], i * 16 + iota16, mask=sel)
        return (nsel + jnp.sum(sel01), neq + jnp.sum(eq01))

    lax.fori_loop(0, _N // 16, _sel, (jnp.int32(0), jnp.int32(0)))

    @pl.when(wid == 0)
    def _():
        pltpu.sync_copy(idx_v, topidx_hbm)

    pltpu.async_copy(h_hbm.at[idx_v.at[pl.ds(base, _RPW)]], xrows_v, xsem).wait()
    pltpu.sync_copy(xrows_v, xpool_hbm.at[pl.ds(base, _RPW)])

    def _mkpad(v, _):
        lane = iota16
        g = jnp.minimum(2 * v + (lane >> 3), jnp.int32(_NB - 1))
        j = jnp.minimum(lane & 7, _RB - 1)
        pad_v[pl.ds(v * 16, 16)] = plsc.load_gather(idx_v, [base + g * _RB + j])
        return 0

    lax.fori_loop(0, (_NB + 2 + 1) // 2, _mkpad, 0)

    isems = (isem0, isem1)
    osems = (osem0, osem1)
    rowbufs = (rows0_v, rows1_v)
    orowbufs = (orow0_v, orow1_v)

    def _pool(tab_hbm, out_hbm):
        def _fire_in(bi, b):
            pltpu.async_copy(tab_hbm.at[pad_v.at[pl.ds(bi * 8, _RB)]],
                             rowbufs[b], isems[b])

        def _slot(g, b, first):
            bi = 2 * g + b
            pltpu.make_async_copy(tab_hbm.at[pad_v.at[pl.ds(bi * 8, _RB)]],
                                  rowbufs[b], isems[b]).wait()
            ob = orowbufs[b]
            if not first:
                pltpu.make_async_copy(
                    ob,
                    out_hbm.at[pl.ds((base + (bi - 2) * _RB) * _K, _RB * _K)],
                    osems[b]).wait()

            def cols(ci, _):
                cvec = idx_v[pl.ds(ci * 16, 16)]
                for r in range(_RB):
                    vals = plsc.load_gather(
                        rowbufs[b], [jnp.full((16,), r, jnp.int32), cvec])
                    ob[pl.ds(r * _K + ci * 16, 16)] = vals
                return 0

            lax.fori_loop(0, _K // 16, cols, 0, unroll=4)
            pltpu.async_copy(
                ob, out_hbm.at[pl.ds((base + bi * _RB) * _K, _RB * _K)],
                osems[b])
            _fire_in(bi + 2, b)
            return bi

        for b in range(2):
            _fire_in(jnp.int32(b), b)
        for b in range(2):
            _slot(jnp.int32(0), b, True)

        def outer(g, _):
            _slot(g, 0, False)
            _slot(g, 1, False)
            return 0

        lax.fori_loop(1, _NB // 2, outer, 0)

        for b in range(2):
            bi = _NB - 2 + b
            pltpu.make_async_copy(
                tab_hbm.at[pad_v.at[pl.ds((bi + 2) * 8, _RB)]],
                rowbufs[b], isems[b]).wait()
            pltpu.make_async_copy(
                orowbufs[b],
                out_hbm.at[pl.ds((base + bi * _RB) * _K, _RB * _K)],
                osems[b]).wait()

    _pool(adj_hbm, adjp_hbm)
    _pool(motif_hbm, motifp_hbm)


def _sc_select_pool(keys, h, adjacency, motif):
    mesh = plsc.VectorSubcoreMesh(core_axis_name="c", subcore_axis_name="s")
    f = pl.kernel(
        _sc_body,
        out_type=(
            jax.ShapeDtypeStruct((_K,), jnp.int32),
            jax.ShapeDtypeStruct((_K, _C), jnp.float32),
            jax.ShapeDtypeStruct((_K * _K,), jnp.float32),
            jax.ShapeDtypeStruct((_K * _K,), jnp.float32),
        ),
        mesh=mesh,
        compiler_params=pltpu.CompilerParams(needs_layout_passes=False),
        scratch_types=[
            pltpu.VMEM((_N,), jnp.int32),
            pltpu.VMEM((_K,), jnp.int32),
            pltpu.VMEM((_RPW, _C), jnp.float32),
            pltpu.VMEM(((_NB + 3) // 2 * 16,), jnp.int32),
            pltpu.VMEM((_RB, _N), jnp.float32),
            pltpu.VMEM((_RB, _N), jnp.float32),
            pltpu.VMEM((_RB * _K,), jnp.float32),
            pltpu.VMEM((_RB * _K,), jnp.float32),
            pltpu.SemaphoreType.DMA,
            pltpu.SemaphoreType.DMA,
            pltpu.SemaphoreType.DMA,
            pltpu.SemaphoreType.DMA,
            pltpu.SemaphoreType.DMA,
        ],
    )
    return f(keys, h, adjacency, motif)


def kernel(x, adjacency, motif_adjacency, W_gcn, b_gcn, W_lin, b_lin):
    num_nodes = x.shape[0]
    A = motif_adjacency + jnp.eye(num_nodes, dtype=x.dtype)
    deg = A.sum(axis=0)
    dinv = jnp.where(deg > 0, 1.0 / jnp.sqrt(deg), 0.0)
    norm = dinv[:, None] * A * dinv[None, :]
    xw = x @ W_gcn
    agg = norm.T @ xw + b_gcn
    h = jnp.tanh(agg)
    scores = (h @ W_lin + b_lin).squeeze(-1)

    bits = lax.bitcast_convert_type(scores, jnp.int32)
    keys = bits ^ ((bits >> 31) & jnp.int32(0x7FFFFFFF))

    top_idx, x_pool, adjp_flat, motifp_flat = _sc_select_pool(
        keys, h, adjacency, motif_adjacency)
    return (x_pool, adjp_flat.reshape(_K, _K), motifp_flat.reshape(_K, _K),
            scores, top_idx)


# speedup vs baseline: 1.1693x; 1.0013x over previous
import math

import jax
import jax.numpy as jnp
import numpy as np
from jax import lax
from jax.experimental import pallas as pl
from jax.experimental.pallas import tpu as pltpu
from jax.experimental.pallas import tpu_sc as plsc

_N = 8192
_C = 128
_K = 4096
_NC = 2
_NS = 16
_NW = _NC * _NS
_RPW = _K // _NW
_RB = 4
_NB = _RPW // _RB
_MININT = np.int32(-2147483648)


def _sc_body(keys_hbm, h_hbm, adj_hbm, motif_hbm,
             topidx_hbm, xpool_hbm, adjp_hbm, motifp_hbm,
             key_v, idx_v, xrows_v, pad_v, rows0_v, rows1_v, orow0_v, orow1_v,
             isem0, isem1, osem0, osem1, xsem):
    cid = lax.axis_index("c")
    sid = lax.axis_index("s")
    wid = sid * _NC + cid
    base = wid * _RPW

    ones16 = jnp.ones((16,), jnp.int32)
    zeros16 = jnp.zeros((16,), jnp.int32)
    iota16 = lax.iota(jnp.int32, 16)

    pltpu.sync_copy(keys_hbm, key_v)

    def _count(ts, strict):
        def body(i, acc):
            for u in range(8):
                mvec = key_v[pl.ds((i * 8 + u) * 16, 16)]
                msk = (mvec > ts) if strict else (mvec >= ts)
                acc = acc + jnp.where(msk, ones16, zeros16)
            return acc
        accv = lax.fori_loop(0, _N // (16 * 8), body, zeros16)
        return jnp.sum(accv)

    def _bis(b, tu):
        cand = tu | (jnp.int32(1) << (jnp.int32(31) - b))
        cnt = _count(cand ^ _MININT, False)
        return jnp.where(cnt >= _K, cand, tu)

    tu = lax.fori_loop(0, 32, _bis, jnp.int32(0))
    ts = tu ^ _MININT
    need = jnp.int32(_K) - _count(ts, True)

    def _sel(i, carry):
        nsel, neq = carry
        mvec = key_v[pl.ds(i * 16, 16)]
        gt = mvec > ts
        eq = mvec == ts
        eq01 = jnp.where(eq, ones16, zeros16)
        eqrank = neq + plsc.cumsum(eq01) - eq01
        sel = gt | (eq & (eqrank < need))
        sel01 = jnp.where(sel, ones16, zeros16)
        pos = nsel + plsc.cumsum(sel01) - sel01
        pos = jnp.where(sel, pos, zeros16)
        plsc.store_scatter(idx_v, [pos], i * 16 + iota16, mask=sel)
        return (nsel + jnp.sum(sel01), neq + jnp.sum(eq01))

    lax.fori_loop(0, _N // 16, _sel, (jnp.int32(0), jnp.int32(0)))

    @pl.when(wid == 0)
    def _():
        pltpu.sync_copy(idx_v, topidx_hbm)

    pltpu.async_copy(h_hbm.at[idx_v.at[pl.ds(base, _RPW)]], xrows_v, xsem).wait()
    pltpu.sync_copy(xrows_v, xpool_hbm.at[pl.ds(base, _RPW)])

    def _mkpad(v, _):
        lane = iota16
        g = jnp.minimum(2 * v + (lane >> 3), jnp.int32(_NB - 1))
        j = jnp.minimum(lane & 7, _RB - 1)
        pad_v[pl.ds(v * 16, 16)] = plsc.load_gather(idx_v, [base + g * _RB + j])
        return 0

    lax.fori_loop(0, (_NB + 2 + 1) // 2, _mkpad, 0)

    isems = (isem0, isem1)
    osems = (osem0, osem1)
    rowbufs = (rows0_v, rows1_v)
    orowbufs = (orow0_v, orow1_v)

    def _pool(tab_hbm, out_hbm):
        def _fire_in(bi, b):
            pltpu.async_copy(tab_hbm.at[pad_v.at[pl.ds(bi * 8, _RB)]],
                             rowbufs[b], isems[b])

        def _slot(g, b, first):
            bi = 2 * g + b
            pltpu.make_async_copy(tab_hbm.at[pad_v.at[pl.ds(bi * 8, _RB)]],
                                  rowbufs[b], isems[b]).wait()
            ob = orowbufs[b]
            if not first:
                pltpu.make_async_copy(
                    ob,
                    out_hbm.at[pl.ds((base + (bi - 2) * _RB) * _K, _RB * _K)],
                    osems[b]).wait()

            def cols(ci, _):
                cvec = idx_v[pl.ds(ci * 16, 16)]
                for r in range(_RB):
                    vals = plsc.load_gather(
                        rowbufs[b], [jnp.full((16,), r, jnp.int32), cvec])
                    ob[pl.ds(r * _K + ci * 16, 16)] = vals
                return 0

            lax.fori_loop(0, _K // 16, cols, 0, unroll=8)
            pltpu.async_copy(
                ob, out_hbm.at[pl.ds((base + bi * _RB) * _K, _RB * _K)],
                osems[b])
            _fire_in(bi + 2, b)
            return bi

        for b in range(2):
            _fire_in(jnp.int32(b), b)
        for b in range(2):
            _slot(jnp.int32(0), b, True)

        def outer(g, _):
            _slot(g, 0, False)
            _slot(g, 1, False)
            return 0

        lax.fori_loop(1, _NB // 2, outer, 0)

        for b in range(2):
            bi = _NB - 2 + b
            pltpu.make_async_copy(
                tab_hbm.at[pad_v.at[pl.ds((bi + 2) * 8, _RB)]],
                rowbufs[b], isems[b]).wait()
            pltpu.make_async_copy(
                orowbufs[b],
                out_hbm.at[pl.ds((base + bi * _RB) * _K, _RB * _K)],
                osems[b]).wait()

    _pool(adj_hbm, adjp_hbm)
    _pool(motif_hbm, motifp_hbm)


def _sc_select_pool(keys, h, adjacency, motif):
    mesh = plsc.VectorSubcoreMesh(core_axis_name="c", subcore_axis_name="s")
    f = pl.kernel(
        _sc_body,
        out_type=(
            jax.ShapeDtypeStruct((_K,), jnp.int32),
            jax.ShapeDtypeStruct((_K, _C), jnp.float32),
            jax.ShapeDtypeStruct((_K * _K,), jnp.float32),
            jax.ShapeDtypeStruct((_K * _K,), jnp.float32),
        ),
        mesh=mesh,
        compiler_params=pltpu.CompilerParams(needs_layout_passes=False),
        scratch_types=[
            pltpu.VMEM((_N,), jnp.int32),
            pltpu.VMEM((_K,), jnp.int32),
            pltpu.VMEM((_RPW, _C), jnp.float32),
            pltpu.VMEM(((_NB + 3) // 2 * 16,), jnp.int32),
            pltpu.VMEM((_RB, _N), jnp.float32),
            pltpu.VMEM((_RB, _N), jnp.float32),
            pltpu.VMEM((_RB * _K,), jnp.float32),
            pltpu.VMEM((_RB * _K,), jnp.float32),
            pltpu.SemaphoreType.DMA,
            pltpu.SemaphoreType.DMA,
            pltpu.SemaphoreType.DMA,
            pltpu.SemaphoreType.DMA,
            pltpu.SemaphoreType.DMA,
        ],
    )
    return f(keys, h, adjacency, motif)


def kernel(x, adjacency, motif_adjacency, W_gcn, b_gcn, W_lin, b_lin):
    num_nodes = x.shape[0]
    A = motif_adjacency + jnp.eye(num_nodes, dtype=x.dtype)
    deg = A.sum(axis=0)
    dinv = jnp.where(deg > 0, 1.0 / jnp.sqrt(deg), 0.0)
    norm = dinv[:, None] * A * dinv[None, :]
    xw = x @ W_gcn
    agg = norm.T @ xw + b_gcn
    h = jnp.tanh(agg)
    scores = (h @ W_lin + b_lin).squeeze(-1)

    bits = lax.bitcast_convert_type(scores, jnp.int32)
    keys = bits ^ ((bits >> 31) & jnp.int32(0x7FFFFFFF))

    top_idx, x_pool, adjp_flat, motifp_flat = _sc_select_pool(
        keys, h, adjacency, motif_adjacency)
    return (x_pool, adjp_flat.reshape(_K, _K), motifp_flat.reshape(_K, _K),
            scores, top_idx)
